# baseline (speedup 1.0000x reference)
import functools

import jax
import jax.numpy as jnp
from jax import lax
from jax.experimental import pallas as pl
from jax.experimental.pallas import tpu as pltpu
from jax.experimental.pallas import tpu_sc as plsc

_T = 100000
_D = 64
_B = 16384

_NC = 2
_NS = 16
_NW = _NC * _NS
_DIMS_PER_W = _D // _NW
_XCHUNK = 4096
_NXCHUNK = _B // _XCHUNK
_L = 16
_UNROLL = 16

_mesh = plsc.VectorSubcoreMesh(core_axis_name="c", subcore_axis_name="s")


@functools.partial(
    pl.kernel,
    mesh=_mesh,
    compiler_params=pltpu.CompilerParams(needs_layout_passes=False),
    out_type=jax.ShapeDtypeStruct((_D, _B), jnp.float32),
    scratch_types=[
        pltpu.VMEM((_T,), jnp.float32),
        pltpu.VMEM((_B,), jnp.int32),
        pltpu.VMEM((_XCHUNK,), jnp.float32),
        pltpu.VMEM((_XCHUNK,), jnp.float32),
        pltpu.SemaphoreType.DMA,
        pltpu.SemaphoreType.DMA,
        pltpu.SemaphoreType.DMA,
        pltpu.SemaphoreType.DMA,
    ],
)
def _pe_gather_t(pet_hbm, x_hbm, outt_hbm, row_v, x_v, oc0_v, oc1_v,
                 rsem, xsem, wsem0, wsem1):
    wid = lax.axis_index("s") * _NC + lax.axis_index("c")
    ocs = (oc0_v, oc1_v)
    wsems = (wsem0, wsem1)

    xcp = pltpu.async_copy(x_hbm, x_v, xsem)
    pending = [None, None]
    for k in range(_DIMS_PER_W):
        d = wid * _DIMS_PER_W + k
        pltpu.async_copy(pet_hbm.at[d], row_v, rsem).wait()
        if k == 0:
            xcp.wait()
        for q in range(_NXCHUNK):
            buf = q % 2
            oc_v = ocs[buf]
            if pending[buf] is not None:
                pending[buf].wait()

            def gather_block(b, _):
                base = q * _XCHUNK + b * (_L * _UNROLL)
                for j in range(_UNROLL):
                    idx16 = x_v[pl.ds(base + j * _L, _L)]
                    oc_v[pl.ds(b * (_L * _UNROLL) + j * _L, _L)] = (
                        plsc.load_gather(row_v, [idx16])
                    )
                return _

            lax.fori_loop(0, _XCHUNK // (_L * _UNROLL), gather_block, None)
            pending[buf] = pltpu.async_copy(
                oc_v, outt_hbm.at[d, pl.ds(q * _XCHUNK, _XCHUNK)], wsems[buf]
            )
    for cp in pending:
        if cp is not None:
            cp.wait()


def kernel(x, pe):
    outt = _pe_gather_t(pe.T, x.astype(jnp.int32))
    return outt.T


# speedup vs baseline: 1.0176x; 1.0176x over previous
import functools

import jax
import jax.numpy as jnp
from jax import lax
from jax.experimental import pallas as pl
from jax.experimental.pallas import tpu as pltpu
from jax.experimental.pallas import tpu_sc as plsc

_T = 100000
_D = 64
_B = 16384

_NC = 2
_NS = 16
_NW = _NC * _NS
_DIMS_PER_W = _D // _NW
_XCHUNK = 4096
_NXCHUNK = _B // _XCHUNK
_L = 16
_UNROLL = 8

_mesh = plsc.VectorSubcoreMesh(core_axis_name="c", subcore_axis_name="s")


@functools.partial(
    pl.kernel,
    mesh=_mesh,
    compiler_params=pltpu.CompilerParams(needs_layout_passes=False),
    out_type=jax.ShapeDtypeStruct((_D, _B), jnp.float32),
    scratch_types=[
        pltpu.VMEM((_T,), jnp.float32),
        pltpu.VMEM((_B,), jnp.int32),
        pltpu.VMEM((_XCHUNK,), jnp.float32),
        pltpu.VMEM((_XCHUNK,), jnp.float32),
        pltpu.SemaphoreType.DMA,
        pltpu.SemaphoreType.DMA,
        pltpu.SemaphoreType.DMA,
        pltpu.SemaphoreType.DMA,
    ],
)
def _pe_gather_t(pet_hbm, x_hbm, outt_hbm, row_v, x_v, oc0_v, oc1_v,
                 rsem, xsem, wsem0, wsem1):
    wid = lax.axis_index("s") * _NC + lax.axis_index("c")
    ocs = (oc0_v, oc1_v)
    wsems = (wsem0, wsem1)

    xcp = pltpu.async_copy(x_hbm, x_v, xsem)
    pending = [None, None]
    for k in range(_DIMS_PER_W):
        d = wid * _DIMS_PER_W + k
        pltpu.async_copy(pet_hbm.at[d], row_v, rsem).wait()
        if k == 0:
            xcp.wait()
        for q in range(_NXCHUNK):
            buf = q % 2
            oc_v = ocs[buf]
            if pending[buf] is not None:
                pending[buf].wait()

            def gather_block(b, _):
                base = q * _XCHUNK + b * (_L * _UNROLL)
                for j in range(_UNROLL):
                    idx16 = x_v[pl.ds(base + j * _L, _L)]
                    oc_v[pl.ds(b * (_L * _UNROLL) + j * _L, _L)] = (
                        plsc.load_gather(row_v, [idx16])
                    )
                return _

            lax.fori_loop(0, _XCHUNK // (_L * _UNROLL), gather_block, None)
            pending[buf] = pltpu.async_copy(
                oc_v, outt_hbm.at[d, pl.ds(q * _XCHUNK, _XCHUNK)], wsems[buf]
            )
    for cp in pending:
        if cp is not None:
            cp.wait()


def kernel(x, pe):
    outt = _pe_gather_t(pe.T, x.astype(jnp.int32))
    return outt.T


# speedup vs baseline: 1.1877x; 1.1671x over previous
import functools

import jax
import jax.numpy as jnp
from jax import lax
from jax.experimental import pallas as pl
from jax.experimental.pallas import tpu as pltpu
from jax.experimental.pallas import tpu_sc as plsc

_T = 100000
_D = 64
_B = 16384

_NC = 2
_NS = 16
_NW = _NC * _NS
_DIMS_PER_W = _D // _NW
_XCHUNK = 4096
_NXCHUNK = _B // _XCHUNK
_L = 16
_UNROLL = 8

_mesh = plsc.VectorSubcoreMesh(core_axis_name="c", subcore_axis_name="s")


@functools.partial(
    pl.kernel,
    mesh=_mesh,
    compiler_params=pltpu.CompilerParams(needs_layout_passes=False),
    out_type=jax.ShapeDtypeStruct((_D, _B), jnp.float32),
    scratch_types=[
        pltpu.VMEM((_T,), jnp.float32),
        pltpu.VMEM((_B,), jnp.int32),
        pltpu.VMEM((_XCHUNK,), jnp.float32),
        pltpu.VMEM((_XCHUNK,), jnp.float32),
        pltpu.SemaphoreType.DMA,
        pltpu.SemaphoreType.DMA,
        pltpu.SemaphoreType.DMA,
        pltpu.SemaphoreType.DMA,
    ],
)
def _pe_gather_t(pet_hbm, x_hbm, outt_hbm, row_v, x_v, oc0_v, oc1_v,
                 rsem, xsem, wsem0, wsem1):
    wid = lax.axis_index("s") * _NC + lax.axis_index("c")
    ocs = (oc0_v, oc1_v)
    wsems = (wsem0, wsem1)

    xcp = pltpu.async_copy(x_hbm, x_v, xsem)
    pending = [None, None]
    for k in range(_DIMS_PER_W):
        d = wid * _DIMS_PER_W + k
        pltpu.async_copy(pet_hbm.at[d], row_v, rsem).wait()
        if k == 0:
            xcp.wait()
        for q in range(_NXCHUNK):
            buf = q % 2
            oc_v = ocs[buf]
            if pending[buf] is not None:
                pending[buf].wait()

            @plsc.parallel_loop(0, _XCHUNK // _L, unroll=_UNROLL)
            def _gather_block(b):
                off = pl.multiple_of(b * _L, _L)
                idx16 = x_v[pl.ds(q * _XCHUNK + off, _L)]
                oc_v[pl.ds(off, _L)] = plsc.load_gather(row_v, [idx16])
            pending[buf] = pltpu.async_copy(
                oc_v, outt_hbm.at[d, pl.ds(q * _XCHUNK, _XCHUNK)], wsems[buf]
            )
    for cp in pending:
        if cp is not None:
            cp.wait()


def kernel(x, pe):
    outt = _pe_gather_t(pe.T, x.astype(jnp.int32))
    return outt.T
